# baseline (speedup 1.0000x reference)
import functools

import jax
import jax.numpy as jnp
from jax import lax
from jax.experimental import pallas as pl
from jax.experimental.pallas import tpu as pltpu
from jax.experimental.pallas import tpu_sc as plsc

B, N, F, H, K = 4, 4096, 128, 64, 27
NS = 512
RANK_CHUNK = 512
BIG_I32 = 2**30

NW = 32
QPT = (B * NS) // NW
CQ = 4
CR = CQ * K
NCHUNK = QPT // CQ


def _prep_body(x_ref, xv_ref, w1_ref, b1_ref, v_ref, bv_ref,
               xs_ref, xs16_ref, xvn_ref, d_ref, dt_ref):
    x = x_ref[0]
    xv = xv_ref[0]

    h = jnp.tanh(jnp.dot(x, w1_ref[...], preferred_element_type=jnp.float32)
                 + b1_ref[...])
    s_col = jax.nn.sigmoid(
        jnp.dot(h, v_ref[...], preferred_element_type=jnp.float32)
        + bv_ref[...])
    xs_ref[0] = s_col
    xs16_ref[0] = jnp.broadcast_to(s_col, (N, 128))
    s_row = jnp.transpose(s_col)

    jj = lax.broadcasted_iota(jnp.int32, (RANK_CHUNK, N), 1)
    ranks = jnp.zeros((1, N), jnp.int32)
    for ci in range(N // RANK_CHUNK):
        sc = s_col[ci * RANK_CHUNK:(ci + 1) * RANK_CHUNK]
        ii = lax.broadcasted_iota(jnp.int32, (RANK_CHUNK, N), 0) \
            + ci * RANK_CHUNK
        before = (sc > s_row) | ((sc == s_row) & (ii < jj))
        ranks = ranks + jnp.sum(jnp.where(before, 1, 0), axis=0, keepdims=True)

    rr = lax.broadcasted_iota(jnp.int32, (NS, N), 0)
    msel = jnp.where(ranks == rr, 1.0, 0.0)

    xvT = jnp.transpose(xv)
    kc = [xvT[c:c + 1, :] for c in range(3)]
    qc = [jnp.sum(msel * kc[c], axis=1, keepdims=True) for c in range(3)]
    xq = jnp.concatenate(qc, axis=1)
    xvn_ref[0] = xq

    k2 = (kc[0] * kc[0] + kc[1] * kc[1]) + kc[2] * kc[2]
    q2 = (qc[0] * qc[0] + qc[1] * qc[1]) + qc[2] * qc[2]
    cross = jnp.dot(xq, xvT, preferred_element_type=jnp.float32)
    d_ref[0] = (q2 + k2) - 2.0 * cross

    kc_col = [xv[:, c:c + 1] for c in range(3)]
    k2_col = (kc_col[0] * kc_col[0] + kc_col[1] * kc_col[1]) \
        + kc_col[2] * kc_col[2]
    q2_row = jnp.transpose(q2)
    crossT = jnp.dot(xv, jnp.transpose(xq),
                     preferred_element_type=jnp.float32)
    dt_ref[0] = (k2_col + q2_row) - 2.0 * crossT


def _knn_extract(dw0, rows, cols):
    hh = rows // 2
    jj = lax.broadcasted_iota(jnp.int32, (hh, cols), 1)
    lane_k = lax.broadcasted_iota(jnp.int32, (hh, 32), 1)
    halves = (dw0[:hh], dw0[hh:])

    def step(k, carry):
        out = []
        for (dw, pidx) in carry:
            mval = jnp.min(dw, axis=1, keepdims=True)
            cand = jnp.where(dw == mval, jj, BIG_I32)
            idx = jnp.min(cand, axis=1, keepdims=True)
            dw = jnp.where(cand == idx, jnp.inf, dw)
            pidx = pidx + jnp.where(lane_k == k, idx, 0)
            out.append((dw, pidx))
        return tuple(out)

    z = jnp.zeros((hh, 32), jnp.int32)
    (_, p0), (_, p1) = lax.fori_loop(
        0, K, step, ((halves[0], z), (halves[1], z)))
    return jnp.concatenate([p0[:, :K], p1[:, :K]], axis=0)


def _pool_body(d_ref, pool_ref, pg_ref):
    pidx = _knn_extract(d_ref[0], NS, N)
    pool_ref[0] = pidx
    pg_ref[0] = pidx + pl.program_id(0) * N


def _unpool_body(dt_ref, unpool_ref):
    unpool_ref[0] = _knn_extract(dt_ref[0], N, NS)


def _sc_attn_body(x_hbm, xs16_hbm, pg_hbm, out_hbm,
                  idx_v, rows_v, srows_v, ob_v, sem):
    wid = lax.axis_index("s") * 2 + lax.axis_index("c")
    qbase = wid * QPT

    pltpu.sync_copy(pg_hbm.at[wid], idx_v)

    def chunk(c, carry):
        cpa = pltpu.async_copy(x_hbm.at[idx_v.at[c]], rows_v, sem)
        cpb = pltpu.async_copy(xs16_hbm.at[idx_v.at[c]], srows_v, sem)
        cpa.wait()
        cpb.wait()
        for qq in range(CQ):
            sv = [srows_v[qq * K + k, pl.ds(0, 16)] for k in range(K)]
            ssum = sv[0]
            for k in range(1, K):
                ssum = ssum + sv[k]
            inv = 1.0 / ssum
            wk = [s * inv for s in sv]
            for seg in range(F // 16):
                acc = wk[0] * rows_v[qq * K, pl.ds(seg * 16, 16)]
                for k in range(1, K):
                    acc = acc + wk[k] * rows_v[qq * K + k, pl.ds(seg * 16, 16)]
                ob_v[qq, pl.ds(seg * 16, 16)] = acc
        pltpu.sync_copy(ob_v, out_hbm.at[pl.ds(qbase + c * CQ, CQ)])
        return carry

    lax.fori_loop(0, NCHUNK, chunk, 0)


_sc_attn = functools.partial(
    pl.kernel,
    out_type=jax.ShapeDtypeStruct((B * NS, F), jnp.float32),
    mesh=plsc.VectorSubcoreMesh(core_axis_name="c", subcore_axis_name="s"),
    scratch_types=[
        pltpu.VMEM((NCHUNK, CR), jnp.int32),
        pltpu.VMEM((CR, F), jnp.float32),
        pltpu.VMEM((CR, 128), jnp.float32),
        pltpu.VMEM((CQ, F), jnp.float32),
        pltpu.SemaphoreType.DMA,
    ],
)(_sc_attn_body)


@jax.jit
def kernel(x, x_v, W1_w, W1_b, V_w, V_b):
    b1 = W1_b.reshape(1, H)
    bv = V_b.reshape(1, 1)

    full = lambda shape: pl.BlockSpec(shape, lambda b: (0,) * len(shape))
    batched = lambda shape: pl.BlockSpec((1,) + shape,
                                         lambda b: (b,) + (0,) * len(shape))
    params = pltpu.CompilerParams(dimension_semantics=("arbitrary",))

    xs, xs16, xvn, d, dt = pl.pallas_call(
        _prep_body,
        grid=(B,),
        in_specs=[batched((N, F)), batched((N, 3)), full((F, H)),
                  full((1, H)), full((H, 1)), full((1, 1))],
        out_specs=[batched((N, 1)), batched((N, 128)), batched((NS, 3)),
                   batched((NS, N)), batched((N, NS))],
        out_shape=[
            jax.ShapeDtypeStruct((B, N, 1), jnp.float32),
            jax.ShapeDtypeStruct((B, N, 128), jnp.float32),
            jax.ShapeDtypeStruct((B, NS, 3), jnp.float32),
            jax.ShapeDtypeStruct((B, NS, N), jnp.float32),
            jax.ShapeDtypeStruct((B, N, NS), jnp.float32),
        ],
        compiler_params=params,
    )(x, x_v, W1_w, b1, V_w, bv)

    pool, pg = pl.pallas_call(
        _pool_body,
        grid=(B,),
        in_specs=[batched((NS, N))],
        out_specs=[batched((NS, K)), batched((NS, K))],
        out_shape=[jax.ShapeDtypeStruct((B, NS, K), jnp.int32),
                   jax.ShapeDtypeStruct((B, NS, K), jnp.int32)],
        compiler_params=params,
    )(d)

    (unpool,) = pl.pallas_call(
        _unpool_body,
        grid=(B,),
        in_specs=[batched((N, NS))],
        out_specs=[batched((N, K))],
        out_shape=[jax.ShapeDtypeStruct((B, N, K), jnp.int32)],
        compiler_params=params,
    )(dt)

    out = _sc_attn(x.reshape(B * N, F),
                   xs16.reshape(B * N, 128),
                   pg.reshape(NW, NCHUNK, CR))
    return out.reshape(B, NS, F), xvn, xs, pool, unpool


# speedup vs baseline: 1.0347x; 1.0347x over previous
import functools

import jax
import jax.numpy as jnp
from jax import lax
from jax.experimental import pallas as pl
from jax.experimental.pallas import tpu as pltpu
from jax.experimental.pallas import tpu_sc as plsc

B, N, F, H, K = 4, 4096, 128, 64, 27
NS = 512
RANK_CHUNK = 512
BIG_I32 = 2**30

NW = 32
QPT = (B * NS) // NW
CQ = 4
CR = CQ * K
NCHUNK = QPT // CQ


def _prep_body(x_ref, xv_ref, w1_ref, b1_ref, v_ref, bv_ref,
               xs_ref, xs16_ref, xvn_ref, d_ref, dt_ref):
    x = x_ref[0]
    xv = xv_ref[0]

    h = jnp.tanh(jnp.dot(x, w1_ref[...], preferred_element_type=jnp.float32)
                 + b1_ref[...])
    s_col = jax.nn.sigmoid(
        jnp.dot(h, v_ref[...], preferred_element_type=jnp.float32)
        + bv_ref[...])
    xs_ref[0] = s_col
    xs16_ref[0] = jnp.broadcast_to(s_col, (N, 128))
    s_row = jnp.transpose(s_col)

    jj = lax.broadcasted_iota(jnp.int32, (RANK_CHUNK, N), 1)
    ranks = jnp.zeros((1, N), jnp.int32)
    for ci in range(N // RANK_CHUNK):
        sc = s_col[ci * RANK_CHUNK:(ci + 1) * RANK_CHUNK]
        ii = lax.broadcasted_iota(jnp.int32, (RANK_CHUNK, N), 0) \
            + ci * RANK_CHUNK
        before = (sc > s_row) | ((sc == s_row) & (ii < jj))
        ranks = ranks + jnp.sum(jnp.where(before, 1, 0), axis=0, keepdims=True)

    rr = lax.broadcasted_iota(jnp.int32, (NS, N), 0)
    msel = jnp.where(ranks == rr, 1.0, 0.0)

    xvT = jnp.transpose(xv)
    kc = [xvT[c:c + 1, :] for c in range(3)]
    qc = [jnp.sum(msel * kc[c], axis=1, keepdims=True) for c in range(3)]
    xq = jnp.concatenate(qc, axis=1)
    xvn_ref[0] = xq

    k2 = (kc[0] * kc[0] + kc[1] * kc[1]) + kc[2] * kc[2]
    q2 = (qc[0] * qc[0] + qc[1] * qc[1]) + qc[2] * qc[2]
    cross = jnp.dot(xq, xvT, preferred_element_type=jnp.float32)
    d_ref[0] = (q2 + k2) - 2.0 * cross

    kc_col = [xv[:, c:c + 1] for c in range(3)]
    k2_col = (kc_col[0] * kc_col[0] + kc_col[1] * kc_col[1]) \
        + kc_col[2] * kc_col[2]
    q2_row = jnp.transpose(q2)
    crossT = jnp.dot(xv, jnp.transpose(xq),
                     preferred_element_type=jnp.float32)
    dt_ref[0] = (k2_col + q2_row) - 2.0 * crossT


def _knn_extract(dw0, rows, cols):
    hh = rows // 2
    jj = lax.broadcasted_iota(jnp.int32, (hh, cols), 1)
    lane_k = lax.broadcasted_iota(jnp.int32, (hh, 32), 1)
    halves = (dw0[:hh], dw0[hh:])

    def step(k, carry):
        out = []
        for (dw, pidx) in carry:
            mval = jnp.min(dw, axis=1, keepdims=True)
            cand = jnp.where(dw == mval, jj, BIG_I32)
            idx = jnp.min(cand, axis=1, keepdims=True)
            dw = jnp.where(jj == idx, jnp.inf, dw)
            pidx = pidx + jnp.where(lane_k == k, idx, 0)
            out.append((dw, pidx))
        return tuple(out)

    z = jnp.zeros((hh, 32), jnp.int32)
    (_, p0), (_, p1) = lax.fori_loop(
        0, K, step, ((halves[0], z), (halves[1], z)))
    return jnp.concatenate([p0[:, :K], p1[:, :K]], axis=0)


def _pool_body(d_ref, pool_ref, pg_ref):
    pidx = _knn_extract(d_ref[0], NS, N)
    pool_ref[0] = pidx
    pg_ref[0] = pidx + pl.program_id(0) * N


def _unpool_body(dt_ref, unpool_ref):
    unpool_ref[0] = _knn_extract(dt_ref[0], N, NS)


def _sc_attn_body(x_hbm, xs16_hbm, pg_hbm, out_hbm,
                  idx_v, rows_a, srows_a, rows_b, srows_b, ob_v,
                  sem_a, sem_b):
    wid = lax.axis_index("s") * 2 + lax.axis_index("c")
    qbase = wid * QPT

    pltpu.sync_copy(pg_hbm.at[wid], idx_v)

    def fire(c, rows, srows, sem):
        pltpu.async_copy(x_hbm.at[idx_v.at[c]], rows, sem)
        pltpu.async_copy(xs16_hbm.at[idx_v.at[c]], srows, sem)

    def drain(c, rows, srows, sem):
        pltpu.make_async_copy(x_hbm.at[idx_v.at[c]], rows, sem).wait()
        pltpu.make_async_copy(xs16_hbm.at[idx_v.at[c]], srows, sem).wait()

    def compute(c, rows_v, srows_v):
        for qq in range(CQ):
            sv = [srows_v[qq * K + k, pl.ds(0, 16)] for k in range(K)]
            ssum = sv[0]
            for k in range(1, K):
                ssum = ssum + sv[k]
            inv = 1.0 / ssum
            wk = [s * inv for s in sv]
            for seg in range(F // 16):
                acc = wk[0] * rows_v[qq * K, pl.ds(seg * 16, 16)]
                for k in range(1, K):
                    acc = acc + wk[k] * rows_v[qq * K + k, pl.ds(seg * 16, 16)]
                ob_v[qq, pl.ds(seg * 16, 16)] = acc
        pltpu.sync_copy(ob_v, out_hbm.at[pl.ds(qbase + c * CQ, CQ)])

    fire(0, rows_a, srows_a, sem_a)

    def pair(i, carry):
        ca = 2 * i
        fire(ca + 1, rows_b, srows_b, sem_b)
        drain(ca, rows_a, srows_a, sem_a)
        compute(ca, rows_a, srows_a)

        @pl.when(i < NCHUNK // 2 - 1)
        def _():
            fire(ca + 2, rows_a, srows_a, sem_a)

        drain(ca + 1, rows_b, srows_b, sem_b)
        compute(ca + 1, rows_b, srows_b)
        return carry

    lax.fori_loop(0, NCHUNK // 2, pair, 0)


_sc_attn = functools.partial(
    pl.kernel,
    out_type=jax.ShapeDtypeStruct((B * NS, F), jnp.float32),
    mesh=plsc.VectorSubcoreMesh(core_axis_name="c", subcore_axis_name="s"),
    scratch_types=[
        pltpu.VMEM((NCHUNK, CR), jnp.int32),
        pltpu.VMEM((CR, F), jnp.float32),
        pltpu.VMEM((CR, 128), jnp.float32),
        pltpu.VMEM((CR, F), jnp.float32),
        pltpu.VMEM((CR, 128), jnp.float32),
        pltpu.VMEM((CQ, F), jnp.float32),
        pltpu.SemaphoreType.DMA,
        pltpu.SemaphoreType.DMA,
    ],
)(_sc_attn_body)


@jax.jit
def kernel(x, x_v, W1_w, W1_b, V_w, V_b):
    b1 = W1_b.reshape(1, H)
    bv = V_b.reshape(1, 1)

    full = lambda shape: pl.BlockSpec(shape, lambda b: (0,) * len(shape))
    batched = lambda shape: pl.BlockSpec((1,) + shape,
                                         lambda b: (b,) + (0,) * len(shape))
    params = pltpu.CompilerParams(dimension_semantics=("arbitrary",))

    xs, xs16, xvn, d, dt = pl.pallas_call(
        _prep_body,
        grid=(B,),
        in_specs=[batched((N, F)), batched((N, 3)), full((F, H)),
                  full((1, H)), full((H, 1)), full((1, 1))],
        out_specs=[batched((N, 1)), batched((N, 128)), batched((NS, 3)),
                   batched((NS, N)), batched((N, NS))],
        out_shape=[
            jax.ShapeDtypeStruct((B, N, 1), jnp.float32),
            jax.ShapeDtypeStruct((B, N, 128), jnp.float32),
            jax.ShapeDtypeStruct((B, NS, 3), jnp.float32),
            jax.ShapeDtypeStruct((B, NS, N), jnp.float32),
            jax.ShapeDtypeStruct((B, N, NS), jnp.float32),
        ],
        compiler_params=params,
    )(x, x_v, W1_w, b1, V_w, bv)

    pool, pg = pl.pallas_call(
        _pool_body,
        grid=(B,),
        in_specs=[batched((NS, N))],
        out_specs=[batched((NS, K)), batched((NS, K))],
        out_shape=[jax.ShapeDtypeStruct((B, NS, K), jnp.int32),
                   jax.ShapeDtypeStruct((B, NS, K), jnp.int32)],
        compiler_params=params,
    )(d)

    (unpool,) = pl.pallas_call(
        _unpool_body,
        grid=(B,),
        in_specs=[batched((N, NS))],
        out_specs=[batched((N, K))],
        out_shape=[jax.ShapeDtypeStruct((B, N, K), jnp.int32)],
        compiler_params=params,
    )(dt)

    out = _sc_attn(x.reshape(B * N, F),
                   xs16.reshape(B * N, 128),
                   pg.reshape(NW, NCHUNK, CR))
    return out.reshape(B, NS, F), xvn, xs, pool, unpool


# speedup vs baseline: 1.0360x; 1.0013x over previous
import functools

import jax
import jax.numpy as jnp
from jax import lax
from jax.experimental import pallas as pl
from jax.experimental.pallas import tpu as pltpu
from jax.experimental.pallas import tpu_sc as plsc

B, N, F, H, K = 4, 4096, 128, 64, 27
NS = 512
RANK_CHUNK = 512
BIG_I32 = 2**30

NW = 32
QPT = (B * NS) // NW
CQ = 4
CR = CQ * K
NCHUNK = QPT // CQ


def _prep_body(x_ref, xv_ref, w1_ref, b1_ref, v_ref, bv_ref,
               xs_ref, xs16_ref, xvn_ref, d_ref, dt_ref):
    x = x_ref[0]
    xv = xv_ref[0]

    h = jnp.tanh(jnp.dot(x, w1_ref[...], preferred_element_type=jnp.float32)
                 + b1_ref[...])
    s_col = jax.nn.sigmoid(
        jnp.dot(h, v_ref[...], preferred_element_type=jnp.float32)
        + bv_ref[...])
    xs_ref[0] = s_col
    xs16_ref[0] = jnp.broadcast_to(s_col, (N, 128))
    s_row = jnp.transpose(s_col)

    jj = lax.broadcasted_iota(jnp.int32, (RANK_CHUNK, N), 1)
    ranks = jnp.zeros((1, N), jnp.int32)
    for ci in range(N // RANK_CHUNK):
        sc = s_col[ci * RANK_CHUNK:(ci + 1) * RANK_CHUNK]
        ii = lax.broadcasted_iota(jnp.int32, (RANK_CHUNK, N), 0) \
            + ci * RANK_CHUNK
        before = (sc > s_row) | ((sc == s_row) & (ii < jj))
        ranks = ranks + jnp.sum(jnp.where(before, 1, 0), axis=0, keepdims=True)

    rr = lax.broadcasted_iota(jnp.int32, (NS, N), 0)
    msel = jnp.where(ranks == rr, 1.0, 0.0)

    xvT = jnp.transpose(xv)
    kc = [xvT[c:c + 1, :] for c in range(3)]
    qc = [jnp.sum(msel * kc[c], axis=1, keepdims=True) for c in range(3)]
    xq = jnp.concatenate(qc, axis=1)
    xvn_ref[0] = xq

    k2 = (kc[0] * kc[0] + kc[1] * kc[1]) + kc[2] * kc[2]
    q2 = (qc[0] * qc[0] + qc[1] * qc[1]) + qc[2] * qc[2]
    cross = jnp.dot(xq, xvT, preferred_element_type=jnp.float32)
    d_ref[0] = (q2 + k2) - 2.0 * cross

    kc_col = [xv[:, c:c + 1] for c in range(3)]
    k2_col = (kc_col[0] * kc_col[0] + kc_col[1] * kc_col[1]) \
        + kc_col[2] * kc_col[2]
    q2_row = jnp.transpose(q2)
    crossT = jnp.dot(xv, jnp.transpose(xq),
                     preferred_element_type=jnp.float32)
    dt_ref[0] = (k2_col + q2_row) - 2.0 * crossT


def _knn_extract(dw0, rows, cols):
    hh = rows // 2
    jj = lax.broadcasted_iota(jnp.int32, (hh, cols), 1)
    lane_k = lax.broadcasted_iota(jnp.int32, (hh, 32), 1)
    halves = (dw0[:hh], dw0[hh:])

    def step(k, carry):
        out = []
        for (dw, pidx) in carry:
            mval = jnp.min(dw, axis=1, keepdims=True)
            cand = jnp.where(dw == mval, jj, BIG_I32)
            idx = jnp.min(cand, axis=1, keepdims=True)
            dw = jnp.where(jj == idx, jnp.inf, dw)
            pidx = pidx + jnp.where(lane_k == k, idx, 0)
            out.append((dw, pidx))
        return tuple(out)

    z = jnp.zeros((hh, 32), jnp.int32)
    (_, p0), (_, p1) = lax.fori_loop(
        0, K, step, ((halves[0], z), (halves[1], z)))
    return jnp.concatenate([p0[:, :K], p1[:, :K]], axis=0)


def _pool_body(d_ref, pool_ref, pg_ref):
    pidx = _knn_extract(d_ref[0], NS, N)
    pool_ref[0] = pidx
    pg_ref[0] = pidx + pl.program_id(0) * N


def _unpool_body(dt_ref, unpool_ref):
    unpool_ref[0] = _knn_extract(dt_ref[0], N, NS)


def _sc_attn_body(x_hbm, xs16_hbm, pg_hbm, out_hbm,
                  idx_v, rows_v, srows_v, ob_v, sem):
    wid = lax.axis_index("s") * 2 + lax.axis_index("c")
    qbase = wid * QPT

    pltpu.sync_copy(pg_hbm.at[wid], idx_v)

    def chunk(c, carry):
        cpa = pltpu.async_copy(x_hbm.at[idx_v.at[c]], rows_v, sem)
        cpb = pltpu.async_copy(xs16_hbm.at[idx_v.at[c]], srows_v, sem)
        cpa.wait()
        cpb.wait()
        for qq in range(CQ):
            sv = [srows_v[qq * K + k, pl.ds(0, 16)] for k in range(K)]
            ssum = sv[0]
            for k in range(1, K):
                ssum = ssum + sv[k]
            inv = 1.0 / ssum
            wk = [s * inv for s in sv]
            for seg in range(F // 16):
                acc = wk[0] * rows_v[qq * K, pl.ds(seg * 16, 16)]
                for k in range(1, K):
                    acc = acc + wk[k] * rows_v[qq * K + k, pl.ds(seg * 16, 16)]
                ob_v[qq, pl.ds(seg * 16, 16)] = acc
        pltpu.sync_copy(ob_v, out_hbm.at[pl.ds(qbase + c * CQ, CQ)])
        return carry

    lax.fori_loop(0, NCHUNK, chunk, 0)


_sc_attn = functools.partial(
    pl.kernel,
    out_type=jax.ShapeDtypeStruct((B * NS, F), jnp.float32),
    mesh=plsc.VectorSubcoreMesh(core_axis_name="c", subcore_axis_name="s"),
    scratch_types=[
        pltpu.VMEM((NCHUNK, CR), jnp.int32),
        pltpu.VMEM((CR, F), jnp.float32),
        pltpu.VMEM((CR, 128), jnp.float32),
        pltpu.VMEM((CQ, F), jnp.float32),
        pltpu.SemaphoreType.DMA,
    ],
)(_sc_attn_body)


@jax.jit
def kernel(x, x_v, W1_w, W1_b, V_w, V_b):
    b1 = W1_b.reshape(1, H)
    bv = V_b.reshape(1, 1)

    full = lambda shape: pl.BlockSpec(shape, lambda b: (0,) * len(shape))
    batched = lambda shape: pl.BlockSpec((1,) + shape,
                                         lambda b: (b,) + (0,) * len(shape))
    params = pltpu.CompilerParams(dimension_semantics=("arbitrary",))

    xs, xs16, xvn, d, dt = pl.pallas_call(
        _prep_body,
        grid=(B,),
        in_specs=[batched((N, F)), batched((N, 3)), full((F, H)),
                  full((1, H)), full((H, 1)), full((1, 1))],
        out_specs=[batched((N, 1)), batched((N, 128)), batched((NS, 3)),
                   batched((NS, N)), batched((N, NS))],
        out_shape=[
            jax.ShapeDtypeStruct((B, N, 1), jnp.float32),
            jax.ShapeDtypeStruct((B, N, 128), jnp.float32),
            jax.ShapeDtypeStruct((B, NS, 3), jnp.float32),
            jax.ShapeDtypeStruct((B, NS, N), jnp.float32),
            jax.ShapeDtypeStruct((B, N, NS), jnp.float32),
        ],
        compiler_params=params,
    )(x, x_v, W1_w, b1, V_w, bv)

    pool, pg = pl.pallas_call(
        _pool_body,
        grid=(B,),
        in_specs=[batched((NS, N))],
        out_specs=[batched((NS, K)), batched((NS, K))],
        out_shape=[jax.ShapeDtypeStruct((B, NS, K), jnp.int32),
                   jax.ShapeDtypeStruct((B, NS, K), jnp.int32)],
        compiler_params=params,
    )(d)

    (unpool,) = pl.pallas_call(
        _unpool_body,
        grid=(B,),
        in_specs=[batched((N, NS))],
        out_specs=[batched((N, K))],
        out_shape=[jax.ShapeDtypeStruct((B, N, K), jnp.int32)],
        compiler_params=params,
    )(dt)

    out = _sc_attn(x.reshape(B * N, F),
                   xs16.reshape(B * N, 128),
                   pg.reshape(NW, NCHUNK, CR))
    return out.reshape(B, NS, F), xvn, xs, pool, unpool
